# Initial kernel scaffold; baseline (speedup 1.0000x reference)
import functools
import math

import jax
import jax.numpy as jnp
import numpy as np
from jax.experimental import pallas as pl
from jax.experimental.pallas import tpu as pltpu

B = 1024
DIM = 512
C = 1211
K = 3
CN = C * K
R = 0.4
WL = 0.3
TOPK = math.ceil(R * CN)

Cp = 1280
W = K * Cp
RB = 128
CT = 256

_INT_MIN = np.uint32(0x80000000).view(np.int32)
_BITS = [np.uint32(1 << b).view(np.int32) for b in range(32)]


def _prep_kernel(p_ref, centers_ref, z_ref):
    p = p_ref[...]
    ssq = jnp.sum(p * p, axis=0, keepdims=True)
    inv = 1.0 / jnp.maximum(jnp.sqrt(ssq), 1e-12)
    c = p * inv
    centers_ref[...] = c
    z_ref[...] = c[:, :Cp] + c[:, Cp:2 * Cp] + c[:, 2 * Cp:]


def _loss_kernel(x_ref, centers_ref, tgt_ref, out_ref):
    x = x_ref[...]
    cen = centers_ref[...]
    s = jax.lax.dot_general(x, cen, (((1,), (0,)), ((), ())),
                            preferred_element_type=jnp.float32)

    iota = jax.lax.broadcasted_iota(jnp.int32, (RB, W), 1)
    plane = (iota >= Cp).astype(jnp.int32) + (iota >= 2 * Cp).astype(jnp.int32)
    cls = iota - Cp * plane
    valid = cls < C
    tgt = tgt_ref[:, 0:1]

    boosted = jnp.where(valid,
                        s + jnp.where(cls == tgt, 100.0, 0.0),
                        -3e38)
    u = jax.lax.bitcast_convert_type(boosted, jnp.int32)
    es = jnp.where(u >= 0, u, u ^ np.int32(0x7FFFFFFF))

    t_u = jnp.zeros((RB, 1), jnp.int32)
    for b in range(31, -1, -1):
        cand_u = t_u | _BITS[b]
        cand_s = cand_u ^ _INT_MIN
        cnt = jnp.sum((es >= cand_s).astype(jnp.int32), axis=1, keepdims=True)
        t_u = jnp.where(cnt >= TOPK, cand_u, t_u)
    t_s = t_u ^ _INT_MIN

    sel = es >= t_s
    masked = jnp.where(sel, s, 0.0)
    logits = masked[:, :Cp] + masked[:, Cp:2 * Cp] + masked[:, 2 * Cp:]

    se = jnp.where(logits != 0.0, jnp.exp(logits), 0.0)
    denom = 1e-8 + jnp.sum(se, axis=1, keepdims=True)
    cls1 = jax.lax.broadcasted_iota(jnp.int32, (RB, Cp), 1)
    texp = jnp.sum(jnp.where(cls1 == tgt, se, 0.0), axis=1, keepdims=True)
    lossrow = -jnp.log(texp / denom + 1e-20)
    partial = jnp.sum(lossrow) * (1.0 / B)

    @pl.when(pl.program_id(0) == 0)
    def _():
        out_ref[0, 0] = 0.0

    out_ref[0, 0] += partial


def _reg_kernel(c_ref, z_ref, out_ref):
    cb = c_ref[...]
    z = z_ref[...]
    cl = jax.lax.dot_general(cb, z, (((0,), (0,)), ((), ())),
                             preferred_element_type=jnp.float32)

    r0 = pl.program_id(0) * CT
    rows = r0 + jax.lax.broadcasted_iota(jnp.int32, (CT, 1), 0)
    plane = (rows >= Cp).astype(jnp.int32) + (rows >= 2 * Cp).astype(jnp.int32)
    ci = rows - Cp * plane
    rvalid = ci < C

    colv = jax.lax.broadcasted_iota(jnp.int32, (CT, Cp), 1)
    clm = jnp.where(colv < C, cl, -3e38)
    m = jnp.max(clm, axis=1, keepdims=True)
    lse = m + jnp.log(jnp.sum(jnp.exp(clm - m), axis=1, keepdims=True))
    diag = jnp.sum(jnp.where(colv == ci, cl, 0.0), axis=1, keepdims=True)
    contrib = jnp.where(rvalid, lse - diag, 0.0)

    @pl.when(pl.program_id(0) == 0)
    def _():
        out_ref[0, 0] = 0.0

    out_ref[0, 0] += jnp.sum(contrib) * (1.0 / CN)


@jax.jit
def kernel(input, proxies, target):
    pr = proxies.reshape(DIM, C, K)
    planes = [jnp.pad(pr[:, :, k], ((0, 0), (0, Cp - C))) for k in range(K)]
    pcat = jnp.concatenate(planes, axis=1)
    tgt2d = jnp.broadcast_to(target[:, None], (B, RB)).astype(jnp.int32)

    centers, z = pl.pallas_call(
        _prep_kernel,
        out_shape=(
            jax.ShapeDtypeStruct((DIM, W), jnp.float32),
            jax.ShapeDtypeStruct((DIM, Cp), jnp.float32),
        ),
    )(pcat)

    loss_cls = pl.pallas_call(
        _loss_kernel,
        grid=(B // RB,),
        in_specs=[
            pl.BlockSpec((RB, DIM), lambda i: (i, 0)),
            pl.BlockSpec((DIM, W), lambda i: (0, 0)),
            pl.BlockSpec((RB, RB), lambda i: (i, 0)),
        ],
        out_specs=pl.BlockSpec((1, 1), lambda i: (0, 0)),
        out_shape=jax.ShapeDtypeStruct((1, 1), jnp.float32),
    )(input, centers, tgt2d)

    reg = pl.pallas_call(
        _reg_kernel,
        grid=(W // CT,),
        in_specs=[
            pl.BlockSpec((DIM, CT), lambda i: (0, i)),
            pl.BlockSpec((DIM, Cp), lambda i: (0, 0)),
        ],
        out_specs=pl.BlockSpec((1, 1), lambda i: (0, 0)),
        out_shape=jax.ShapeDtypeStruct((1, 1), jnp.float32),
    )(pcat, z)

    return loss_cls[0, 0] + WL * reg[0, 0]


# speedup vs baseline: 44.5634x; 44.5634x over previous
import functools
import math

import jax
import jax.numpy as jnp
import numpy as np
from jax.experimental import pallas as pl
from jax.experimental.pallas import tpu as pltpu

B = 1024
DIM = 512
C = 1211
K = 3
CN = C * K
R = 0.4
WL = 0.3
TOPK = math.ceil(R * CN)

Cp = 1280
W = K * Cp
RB = 128
CT = 256

_INT_MIN = np.uint32(0x80000000).view(np.int32)
_BITS = [np.uint32(1 << b).view(np.int32) for b in range(32)]


def _prep_kernel(p_ref, centers_ref, z_ref):
    p = p_ref[...]
    ssq = jnp.sum(p * p, axis=0, keepdims=True)
    inv = 1.0 / jnp.maximum(jnp.sqrt(ssq), 1e-12)
    c = p * inv
    centers_ref[...] = c
    z_ref[...] = c[:, :Cp] + c[:, Cp:2 * Cp] + c[:, 2 * Cp:]


def _loss_kernel(x_ref, centers_ref, tgt_ref, out_ref):
    x = x_ref[...]
    cen = centers_ref[...]
    s = jax.lax.dot_general(x, cen, (((1,), (0,)), ((), ())),
                            preferred_element_type=jnp.float32)

    iota = jax.lax.broadcasted_iota(jnp.int32, (RB, W), 1)
    plane = (iota >= Cp).astype(jnp.int32) + (iota >= 2 * Cp).astype(jnp.int32)
    cls = iota - Cp * plane
    valid = cls < C
    tgt = tgt_ref[:, 0:1]

    boosted = jnp.where(valid,
                        s + jnp.where(cls == tgt, 100.0, 0.0),
                        -3e38)
    u = jax.lax.bitcast_convert_type(boosted, jnp.int32)
    es = jnp.where(u >= 0, u, u ^ np.int32(0x7FFFFFFF))

    t_u = jnp.zeros((RB, 1), jnp.int32)
    for b in range(31, -1, -1):
        cand_u = t_u | _BITS[b]
        cand_s = cand_u ^ _INT_MIN
        cnt = jnp.sum((es >= cand_s).astype(jnp.int32), axis=1, keepdims=True)
        t_u = jnp.where(cnt >= TOPK, cand_u, t_u)
    t_s = t_u ^ _INT_MIN

    sel = es >= t_s
    masked = jnp.where(sel, s, 0.0)
    logits = masked[:, :Cp] + masked[:, Cp:2 * Cp] + masked[:, 2 * Cp:]

    se = jnp.where(logits != 0.0, jnp.exp(logits), 0.0)
    denom = 1e-8 + jnp.sum(se, axis=1, keepdims=True)
    cls1 = jax.lax.broadcasted_iota(jnp.int32, (RB, Cp), 1)
    texp = jnp.sum(jnp.where(cls1 == tgt, se, 0.0), axis=1, keepdims=True)
    lossrow = -jnp.log(texp / denom + 1e-20)
    partial = jnp.sum(lossrow, keepdims=True).reshape(1, 1) * (1.0 / B)

    @pl.when(pl.program_id(0) == 0)
    def _():
        out_ref[...] = jnp.zeros((1, 1), jnp.float32)

    out_ref[...] += partial


def _reg_kernel(c_ref, z_ref, out_ref):
    cb = c_ref[...]
    z = z_ref[...]
    cl = jax.lax.dot_general(cb, z, (((0,), (0,)), ((), ())),
                             preferred_element_type=jnp.float32)

    r0 = pl.program_id(0) * CT
    rows = r0 + jax.lax.broadcasted_iota(jnp.int32, (CT, 1), 0)
    plane = (rows >= Cp).astype(jnp.int32) + (rows >= 2 * Cp).astype(jnp.int32)
    ci = rows - Cp * plane
    rvalid = ci < C

    colv = jax.lax.broadcasted_iota(jnp.int32, (CT, Cp), 1)
    clm = jnp.where(colv < C, cl, -3e38)
    m = jnp.max(clm, axis=1, keepdims=True)
    lse = m + jnp.log(jnp.sum(jnp.exp(clm - m), axis=1, keepdims=True))
    diag = jnp.sum(jnp.where(colv == ci, cl, 0.0), axis=1, keepdims=True)
    contrib = jnp.where(rvalid, lse - diag, 0.0)

    @pl.when(pl.program_id(0) == 0)
    def _():
        out_ref[...] = jnp.zeros((1, 1), jnp.float32)

    out_ref[...] += jnp.sum(contrib, keepdims=True).reshape(1, 1) * (1.0 / CN)


@jax.jit
def kernel(input, proxies, target):
    pr = proxies.reshape(DIM, C, K)
    planes = [jnp.pad(pr[:, :, k], ((0, 0), (0, Cp - C))) for k in range(K)]
    pcat = jnp.concatenate(planes, axis=1)
    tgt2d = jnp.broadcast_to(target[:, None], (B, RB)).astype(jnp.int32)

    centers, z = pl.pallas_call(
        _prep_kernel,
        out_shape=(
            jax.ShapeDtypeStruct((DIM, W), jnp.float32),
            jax.ShapeDtypeStruct((DIM, Cp), jnp.float32),
        ),
    )(pcat)

    loss_cls = pl.pallas_call(
        _loss_kernel,
        grid=(B // RB,),
        in_specs=[
            pl.BlockSpec((RB, DIM), lambda i: (i, 0)),
            pl.BlockSpec((DIM, W), lambda i: (0, 0)),
            pl.BlockSpec((RB, RB), lambda i: (i, 0)),
        ],
        out_specs=pl.BlockSpec((1, 1), lambda i: (0, 0)),
        out_shape=jax.ShapeDtypeStruct((1, 1), jnp.float32),
    )(input, centers, tgt2d)

    reg = pl.pallas_call(
        _reg_kernel,
        grid=(W // CT,),
        in_specs=[
            pl.BlockSpec((DIM, CT), lambda i: (0, i)),
            pl.BlockSpec((DIM, Cp), lambda i: (0, 0)),
        ],
        out_specs=pl.BlockSpec((1, 1), lambda i: (0, 0)),
        out_shape=jax.ShapeDtypeStruct((1, 1), jnp.float32),
    )(centers, z)

    return loss_cls[0, 0] + WL * reg[0, 0]


# speedup vs baseline: 50.7883x; 1.1397x over previous
import functools
import math

import jax
import jax.numpy as jnp
import numpy as np
from jax.experimental import pallas as pl
from jax.experimental.pallas import tpu as pltpu

B = 1024
DIM = 512
C = 1211
K = 3
CN = C * K
R = 0.4
WL = 0.3
TOPK = math.ceil(R * CN)

Cp = 1280
W = K * Cp
RB = 128
CT = 256

_INT_MIN = np.uint32(0x80000000).view(np.int32)
_BITS = [np.uint32(1 << b).view(np.int32) for b in range(32)]


def _prep_kernel(p_ref, centers_ref, z_ref):
    p = p_ref[...]
    ssq = jnp.sum(p * p, axis=0, keepdims=True)
    inv = 1.0 / jnp.maximum(jnp.sqrt(ssq), 1e-12)
    c = p * inv
    centers_ref[...] = c
    z_ref[...] = c[:, :Cp] + c[:, Cp:2 * Cp] + c[:, 2 * Cp:]


def _loss_kernel(x_ref, centers_ref, tgt_ref, out_ref):
    x = x_ref[...]
    cen = centers_ref[...]
    s = jax.lax.dot_general(x, cen, (((1,), (0,)), ((), ())),
                            preferred_element_type=jnp.float32)

    iota = jax.lax.broadcasted_iota(jnp.int32, (RB, W), 1)
    plane = (iota >= Cp).astype(jnp.int32) + (iota >= 2 * Cp).astype(jnp.int32)
    cls = iota - Cp * plane
    valid = cls < C
    tgt = tgt_ref[:, 0:1]

    boosted = jnp.where(valid,
                        s + jnp.where(cls == tgt, 100.0, 0.0),
                        -3e38)
    u = jax.lax.bitcast_convert_type(boosted, jnp.int32)
    es = jnp.where(u >= 0, u, u ^ np.int32(0x7FFFFFFF))

    topk16 = np.int16(TOPK)
    bias16 = np.uint16(0x8000).view(np.int16)

    def count16(mask):
        ones = jnp.where(mask, jnp.bfloat16(1), jnp.bfloat16(0))
        acc = ones[:, :128]
        for i in range(1, W // 128):
            acc = acc + ones[:, i * 128:(i + 1) * 128]
        cnt = jnp.sum(acc.astype(jnp.float32), axis=1, keepdims=True)
        return cnt.astype(jnp.int32).astype(jnp.int16)

    hi = (es >> 16).astype(jnp.int16)
    t_hi_u = jnp.zeros((RB, 1), jnp.int16)
    for b in range(15, -1, -1):
        cand_u = t_hi_u | np.uint16(1 << b).view(np.int16)
        cand_s = cand_u ^ bias16
        cnt = count16(hi >= cand_s)
        t_hi_u = jnp.where(cnt >= topk16, cand_u, t_hi_u)
    t_hi = t_hi_u ^ bias16

    c_eq = count16(hi == t_hi)
    c_ge = count16(hi >= t_hi)
    kk2 = topk16 - (c_ge - c_eq)

    lo16 = ((es & np.int32(0xFFFF)) - 32768).astype(jnp.int16)
    act = jnp.where(hi == t_hi, lo16, np.int16(-32768))
    t_lo_u = jnp.zeros((RB, 1), jnp.int16)
    for b in range(15, -1, -1):
        cand_u = t_lo_u | np.uint16(1 << b).view(np.int16)
        cand_s = cand_u ^ bias16
        cnt = count16(act >= cand_s)
        t_lo_u = jnp.where(cnt >= kk2, cand_u, t_lo_u)

    t_s = ((t_hi.astype(jnp.int32)) << 16) | (t_lo_u.astype(jnp.int32) & 0xFFFF)

    sel = es >= t_s
    masked = jnp.where(sel, s, 0.0)
    logits = masked[:, :Cp] + masked[:, Cp:2 * Cp] + masked[:, 2 * Cp:]

    se = jnp.where(logits != 0.0, jnp.exp(logits), 0.0)
    denom = 1e-8 + jnp.sum(se, axis=1, keepdims=True)
    cls1 = jax.lax.broadcasted_iota(jnp.int32, (RB, Cp), 1)
    texp = jnp.sum(jnp.where(cls1 == tgt, se, 0.0), axis=1, keepdims=True)
    lossrow = -jnp.log(texp / denom + 1e-20)
    partial = jnp.sum(lossrow, keepdims=True).reshape(1, 1) * (1.0 / B)

    @pl.when(pl.program_id(0) == 0)
    def _():
        out_ref[...] = jnp.zeros((1, 1), jnp.float32)

    out_ref[...] += partial


def _reg_kernel(c_ref, z_ref, out_ref):
    cb = c_ref[...]
    z = z_ref[...]
    cl = jax.lax.dot_general(cb, z, (((0,), (0,)), ((), ())),
                             preferred_element_type=jnp.float32)

    r0 = pl.program_id(0) * CT
    rows = r0 + jax.lax.broadcasted_iota(jnp.int32, (CT, 1), 0)
    plane = (rows >= Cp).astype(jnp.int32) + (rows >= 2 * Cp).astype(jnp.int32)
    ci = rows - Cp * plane
    rvalid = ci < C

    colv = jax.lax.broadcasted_iota(jnp.int32, (CT, Cp), 1)
    clm = jnp.where(colv < C, cl, -3e38)
    m = jnp.max(clm, axis=1, keepdims=True)
    lse = m + jnp.log(jnp.sum(jnp.exp(clm - m), axis=1, keepdims=True))
    diag = jnp.sum(jnp.where(colv == ci, cl, 0.0), axis=1, keepdims=True)
    contrib = jnp.where(rvalid, lse - diag, 0.0)

    @pl.when(pl.program_id(0) == 0)
    def _():
        out_ref[...] = jnp.zeros((1, 1), jnp.float32)

    out_ref[...] += jnp.sum(contrib, keepdims=True).reshape(1, 1) * (1.0 / CN)


@jax.jit
def kernel(input, proxies, target):
    pr = proxies.reshape(DIM, C, K)
    planes = [jnp.pad(pr[:, :, k], ((0, 0), (0, Cp - C))) for k in range(K)]
    pcat = jnp.concatenate(planes, axis=1)
    tgt2d = jnp.broadcast_to(target[:, None], (B, RB)).astype(jnp.int32)

    centers, z = pl.pallas_call(
        _prep_kernel,
        out_shape=(
            jax.ShapeDtypeStruct((DIM, W), jnp.float32),
            jax.ShapeDtypeStruct((DIM, Cp), jnp.float32),
        ),
    )(pcat)

    loss_cls = pl.pallas_call(
        _loss_kernel,
        grid=(B // RB,),
        in_specs=[
            pl.BlockSpec((RB, DIM), lambda i: (i, 0)),
            pl.BlockSpec((DIM, W), lambda i: (0, 0)),
            pl.BlockSpec((RB, RB), lambda i: (i, 0)),
        ],
        out_specs=pl.BlockSpec((1, 1), lambda i: (0, 0)),
        out_shape=jax.ShapeDtypeStruct((1, 1), jnp.float32),
    )(input, centers, tgt2d)

    reg = pl.pallas_call(
        _reg_kernel,
        grid=(W // CT,),
        in_specs=[
            pl.BlockSpec((DIM, CT), lambda i: (0, i)),
            pl.BlockSpec((DIM, Cp), lambda i: (0, 0)),
        ],
        out_specs=pl.BlockSpec((1, 1), lambda i: (0, 0)),
        out_shape=jax.ShapeDtypeStruct((1, 1), jnp.float32),
    )(centers, z)

    return loss_cls[0, 0] + WL * reg[0, 0]
